# baseline (speedup 1.0000x reference)
import functools

import jax
import jax.numpy as jnp
from jax import lax
from jax.experimental import pallas as pl
from jax.experimental.pallas import tpu as pltpu
from jax.experimental.pallas import tpu_sc as plsc

_NUM_ITEM = 1000000
_FEAT_DIM = 32
_BATCH = 16384

_info = plsc.get_sparse_core_info()
_NC = _info.num_cores
_NS = _info.num_subcores
_NW = _NC * _NS
_B_PER_W = _BATCH // _NW

_mesh = plsc.VectorSubcoreMesh(core_axis_name="c", subcore_axis_name="s")


@functools.partial(
    pl.kernel,
    mesh=_mesh,
    out_type=(
        jax.ShapeDtypeStruct((_BATCH, _FEAT_DIM), jnp.float32),
        jax.ShapeDtypeStruct((_BATCH, _FEAT_DIM), jnp.float32),
    ),
    scratch_types=[
        pltpu.VMEM((_B_PER_W,), jnp.int32),
        pltpu.VMEM((_B_PER_W, _FEAT_DIM), jnp.float32),
        pltpu.VMEM((_B_PER_W, _FEAT_DIM), jnp.float32),
        pltpu.SemaphoreType.DMA,
        pltpu.SemaphoreType.DMA,
    ],
    compiler_params=pltpu.CompilerParams(use_tc_tiling_on_sc=False),
)
def _gather2(idx_hbm, mu_hbm, lv_hbm, mu_out, lv_out,
             idx_v, mu_v, lv_v, sem_mu, sem_lv):
    wid = lax.axis_index("s") * _NC + lax.axis_index("c")
    base = wid * _B_PER_W
    pltpu.sync_copy(idx_hbm.at[pl.ds(base, _B_PER_W)], idx_v)
    cp_mu = pltpu.async_copy(mu_hbm.at[idx_v], mu_v, sem_mu)
    cp_lv = pltpu.async_copy(lv_hbm.at[idx_v], lv_v, sem_lv)
    cp_mu.wait()
    pltpu.sync_copy(mu_v, mu_out.at[pl.ds(base, _B_PER_W)])
    cp_lv.wait()
    pltpu.sync_copy(lv_v, lv_out.at[pl.ds(base, _B_PER_W)])


def kernel(item_index, mu_table, logvar_table):
    idx = jnp.squeeze(item_index, axis=1)
    mu, logvar = _gather2(idx, mu_table, logvar_table)
    return (mu, logvar)


# speedup vs baseline: 3.1445x; 3.1445x over previous
import functools

import jax
import jax.numpy as jnp
from jax import lax
from jax.experimental import pallas as pl
from jax.experimental.pallas import tpu as pltpu
from jax.experimental.pallas import tpu_sc as plsc

_NUM_ITEM = 1000000
_FEAT_DIM = 32
_BATCH = 16384

_info = plsc.get_sparse_core_info()
_NC = _info.num_cores
_NS = _info.num_subcores
_NW = _NC * _NS
_B_PER_W = _BATCH // _NW
_GRP = 8
_LANES = 128
_MAX_BASE = _NUM_ITEM - _LANES

_mesh = plsc.VectorSubcoreMesh(core_axis_name="c", subcore_axis_name="s")


@functools.partial(
    pl.kernel,
    mesh=_mesh,
    out_type=(
        jax.ShapeDtypeStruct((_BATCH // 4, _LANES), jnp.float32),
        jax.ShapeDtypeStruct((_BATCH // 4, _LANES), jnp.float32),
    ),
    scratch_types=[
        pltpu.VMEM((_B_PER_W,), jnp.int32),
        pltpu.VMEM((_GRP, _FEAT_DIM, _LANES), jnp.float32),
        pltpu.VMEM((_GRP, _FEAT_DIM, _LANES), jnp.float32),
        pltpu.VMEM((_B_PER_W * _FEAT_DIM // _LANES, _LANES), jnp.float32),
        pltpu.VMEM((_B_PER_W * _FEAT_DIM // _LANES, _LANES), jnp.float32),
        pltpu.SemaphoreType.DMA,
        pltpu.SemaphoreType.DMA,
    ],
    compiler_params=pltpu.CompilerParams(disable_bounds_checks=True,
                                         needs_layout_passes=False),
)
def _gather2(idx_hbm, mu_hbm, lv_hbm, mu_out, lv_out,
             idx_v, mu_b, lv_b, mu_st, lv_st, sem_mu, sem_lv):
    wid = lax.axis_index("s") * _NC + lax.axis_index("c")
    base = wid * _B_PER_W
    pltpu.sync_copy(idx_hbm.at[pl.ds(base, _B_PER_W)], idx_v)
    iota = lax.iota(jnp.int32, 16)
    iota_hi = iota + 16

    def body(k, _):
        v = idx_v[pl.ds(k * 16, 16)]
        for h in range(2):
            cols = []
            copies = []
            for j in range(_GRP):
                c = v[h * _GRP + j]
                col = pl.multiple_of(lax.bitwise_and(c, jnp.int32(-_LANES)),
                                     _LANES)
                cols.append(col)
                copies.append(pltpu.async_copy(
                    mu_hbm.at[:, pl.ds(col, _LANES)], mu_b.at[j], sem_mu))
                copies.append(pltpu.async_copy(
                    lv_hbm.at[:, pl.ds(col, _LANES)], lv_b.at[j], sem_lv))
            for cp in copies:
                cp.wait()
            for j in range(_GRP):
                c = v[h * _GRP + j]
                lane = jnp.full((16,), c - cols[j], jnp.int32)
                cpos = k * 16 + h * _GRP + j
                row = lax.div(cpos, jnp.int32(4))
                lo = lax.mul(lax.rem(cpos, jnp.int32(4)), jnp.int32(32))
                for st, bufs in ((mu_st, mu_b), (lv_st, lv_b)):
                    r0 = plsc.load_gather(bufs.at[j], [iota, lane])
                    r1 = plsc.load_gather(bufs.at[j], [iota_hi, lane])
                    st[row, pl.ds(lo, 16)] = r0
                    st[row, pl.ds(lo + 16, 16)] = r1
        return 0

    lax.fori_loop(0, _B_PER_W // 16, body, 0)
    rows = _B_PER_W * _FEAT_DIM // _LANES
    pltpu.sync_copy(mu_st, mu_out.at[pl.ds(wid * rows, rows), :])
    pltpu.sync_copy(lv_st, lv_out.at[pl.ds(wid * rows, rows), :])


def kernel(item_index, mu_table, logvar_table):
    idx = jnp.squeeze(item_index, axis=1)
    mu4, lv4 = _gather2(idx, mu_table.T, logvar_table.T)
    return (mu4.reshape(_BATCH, _FEAT_DIM), lv4.reshape(_BATCH, _FEAT_DIM))


# speedup vs baseline: 3.7558x; 1.1944x over previous
import functools

import jax
import jax.numpy as jnp
from jax import lax
from jax.experimental import pallas as pl
from jax.experimental.pallas import tpu as pltpu
from jax.experimental.pallas import tpu_sc as plsc

_NUM_ITEM = 1000000
_FEAT_DIM = 32
_BATCH = 16384

_info = plsc.get_sparse_core_info()
_NC = _info.num_cores
_NS = _info.num_subcores
_NW = _NC * _NS
_B_PER_W = _BATCH // _NW
_G = 4
_LANES = 128

_mesh = plsc.VectorSubcoreMesh(core_axis_name="c", subcore_axis_name="s")


@functools.partial(
    pl.kernel,
    mesh=_mesh,
    out_type=(
        jax.ShapeDtypeStruct((_FEAT_DIM, _BATCH), jnp.float32),
        jax.ShapeDtypeStruct((_FEAT_DIM, _BATCH), jnp.float32),
    ),
    scratch_types=[
        pltpu.VMEM((_B_PER_W,), jnp.int32),
        pltpu.VMEM((2 * _G, _FEAT_DIM, _LANES), jnp.float32),
        pltpu.VMEM((2 * _G, _FEAT_DIM, _LANES), jnp.float32),
        pltpu.VMEM((_FEAT_DIM, _B_PER_W), jnp.float32),
        pltpu.VMEM((_FEAT_DIM, _B_PER_W), jnp.float32),
        pltpu.SemaphoreType.DMA,
        pltpu.SemaphoreType.DMA,
    ],
    compiler_params=pltpu.CompilerParams(disable_bounds_checks=True,
                                         needs_layout_passes=False),
)
def _gather2(idx_hbm, mu_hbm, lv_hbm, mu_out, lv_out,
             idx_v, mu_b, lv_b, mu_st, lv_st, sem_mu, sem_lv):
    wid = lax.axis_index("s") * _NC + lax.axis_index("c")
    base = wid * _B_PER_W
    pltpu.sync_copy(idx_hbm.at[pl.ds(base, _B_PER_W)], idx_v)
    iota = lax.iota(jnp.int32, 16)
    iota_hi = iota + 16

    def issue(v, q, bank):
        copies = []
        for j in range(_G):
            c = v[q * _G + j]
            col = pl.multiple_of(lax.bitwise_and(c, jnp.int32(-_LANES)),
                                 _LANES)
            slot = bank * _G + j
            copies.append(pltpu.async_copy(
                mu_hbm.at[:, pl.ds(col, _LANES)], mu_b.at[slot], sem_mu))
            copies.append(pltpu.async_copy(
                lv_hbm.at[:, pl.ds(col, _LANES)], lv_b.at[slot], sem_lv))
        return copies

    def extract(v, q, bank, k):
        for j in range(_G):
            c = v[q * _G + j]
            lane = jnp.full((16,), lax.bitwise_and(c, jnp.int32(_LANES - 1)),
                            jnp.int32)
            cpos = jnp.full((16,), k * 16 + q * _G + j, jnp.int32)
            slot = bank * _G + j
            for st, bufs in ((mu_st, mu_b), (lv_st, lv_b)):
                r0 = plsc.load_gather(bufs.at[slot], [iota, lane])
                r1 = plsc.load_gather(bufs.at[slot], [iota_hi, lane])
                plsc.store_scatter(st, [iota, cpos], r0)
                plsc.store_scatter(st, [iota_hi, cpos], r1)

    def body(k, _):
        v = idx_v[pl.ds(k * 16, 16)]
        cps = [issue(v, 0, 0), issue(v, 1, 1)]
        for q in range(4):
            for cp in cps[q]:
                cp.wait()
            extract(v, q, q % 2, k)
            if q + 2 < 4:
                cps.append(issue(v, q + 2, q % 2))
        return 0

    lax.fori_loop(0, _B_PER_W // 16, body, 0)
    pltpu.sync_copy(mu_st, mu_out.at[:, pl.ds(base, _B_PER_W)])
    pltpu.sync_copy(lv_st, lv_out.at[:, pl.ds(base, _B_PER_W)])


def kernel(item_index, mu_table, logvar_table):
    idx = jnp.squeeze(item_index, axis=1)
    mu_t, lv_t = _gather2(idx, mu_table.T, logvar_table.T)
    return (mu_t.T, lv_t.T)
